# Initial kernel scaffold; baseline (speedup 1.0000x reference)
import functools

import jax
import jax.numpy as jnp
from jax import lax
from jax.experimental import pallas as pl
from jax.experimental.pallas import tpu as pltpu
from jax.experimental.pallas import tpu_sc as plsc

NC = 2
NS = 16
NW = NC * NS

CHUNK = 2560


def _gather_body(nch, idx_hbm, table_hbm, out_hbm, idx_v, rows_v, sem):
    wid = lax.axis_index("s") * NC + lax.axis_index("c")
    base = wid * (nch * CHUNK)

    def step(i, carry):
        off = base + i * CHUNK
        pltpu.sync_copy(idx_hbm.at[pl.ds(off, CHUNK)], idx_v)
        pltpu.async_copy(table_hbm.at[idx_v], rows_v, sem).wait()
        pltpu.sync_copy(rows_v, out_hbm.at[pl.ds(off, CHUNK)])
        return carry

    lax.fori_loop(0, nch, step, 0)


@functools.partial(jax.jit, static_argnames=("n", "d"))
def _gather(flat_idx, table, n, d):
    nch = n // (NW * CHUNK)
    mesh = plsc.VectorSubcoreMesh(core_axis_name="c", subcore_axis_name="s")
    return pl.kernel(
        functools.partial(_gather_body, nch),
        out_type=jax.ShapeDtypeStruct((n, d), jnp.float32),
        mesh=mesh,
        scratch_types=[
            pltpu.VMEM((CHUNK,), jnp.int32),
            pltpu.VMEM((CHUNK, d), jnp.float32),
            pltpu.SemaphoreType.DMA,
        ],
    )(flat_idx, table)


def kernel(action_idx, table):
    b, h = action_idx.shape
    n = b * h
    d = table.shape[1]
    flat_idx = action_idx.reshape(n).astype(jnp.int32)
    out = _gather(flat_idx, table, n, d)
    return out.reshape(b, h, d)


# speedup vs baseline: 1.1078x; 1.1078x over previous
import functools

import jax
import jax.numpy as jnp
from jax import lax
from jax.experimental import pallas as pl
from jax.experimental.pallas import tpu as pltpu
from jax.experimental.pallas import tpu_sc as plsc

NC = 2
NS = 16
NW = NC * NS

CHUNK = 2560


def _gather_body(nch, idx_hbm, table_hbm, out_hbm, idx_v, rows_v, sem):
    wid = lax.axis_index("s") * NC + lax.axis_index("c")
    base = wid * (nch * CHUNK)

    def step(i, carry):
        off = base + i * CHUNK
        pltpu.sync_copy(idx_hbm.at[pl.ds(off, CHUNK)], idx_v)
        pltpu.async_copy(table_hbm.at[idx_v], rows_v, sem).wait()
        pltpu.sync_copy(rows_v, out_hbm.at[pl.ds(off, CHUNK)])
        return carry

    lax.fori_loop(0, nch, step, 0)


@functools.partial(jax.jit, static_argnames=("n", "d"))
def _gather(flat_idx, table, n, d):
    nch = n // (NW * CHUNK)
    mesh = plsc.VectorSubcoreMesh(core_axis_name="c", subcore_axis_name="s")
    return pl.kernel(
        functools.partial(_gather_body, nch),
        out_type=jax.ShapeDtypeStruct((n, d), jnp.float32),
        mesh=mesh,
        scratch_types=[
            pltpu.VMEM((CHUNK,), jnp.int32),
            pltpu.VMEM((CHUNK, d), jnp.float32),
            pltpu.SemaphoreType.DMA,
        ],
        compiler_params=pltpu.CompilerParams(use_tc_tiling_on_sc=False),
    )(flat_idx, table)


def kernel(action_idx, table):
    b, h = action_idx.shape
    n = b * h
    d = table.shape[1]
    flat_idx = action_idx.reshape(n).astype(jnp.int32)
    out = _gather(flat_idx, table, n, d)
    return out.reshape(b, h, d)


# speedup vs baseline: 1.1113x; 1.0032x over previous
import functools

import jax
import jax.numpy as jnp
from jax import lax
from jax.experimental import pallas as pl
from jax.experimental.pallas import tpu as pltpu
from jax.experimental.pallas import tpu_sc as plsc

NC = 2
NS = 16
NW = NC * NS

CHUNK = 1600


def _gather_body(nch, idx_hbm, table_hbm, out_hbm,
                 idx0, idx1, rows0, rows1,
                 isem0, isem1, gsem0, gsem1, osem0, osem1):
    wid = lax.axis_index("s") * NC + lax.axis_index("c")
    base = wid * (nch * CHUNK)
    idx_v = (idx0, idx1)
    rows_v = (rows0, rows1)
    isem = (isem0, isem1)
    gsem = (gsem0, gsem1)
    osem = (osem0, osem1)

    def idx_load(i):
        return pltpu.async_copy(
            idx_hbm.at[pl.ds(base + i * CHUNK, CHUNK)], idx_v[i % 2], isem[i % 2])

    def gather(i):
        return pltpu.async_copy(table_hbm.at[idx_v[i % 2]], rows_v[i % 2], gsem[i % 2])

    def scatter(i):
        return pltpu.async_copy(
            rows_v[i % 2], out_hbm.at[pl.ds(base + i * CHUNK, CHUNK)], osem[i % 2])

    d_idx, d_g, d_o = {}, {}, {}
    d_idx[0] = idx_load(0)
    if nch > 1:
        d_idx[1] = idx_load(1)
    d_idx[0].wait()
    d_g[0] = gather(0)
    for i in range(nch):
        if i + 1 < nch:
            d_idx[i + 1].wait()
            if i >= 1:
                d_o[i - 1].wait()
            d_g[i + 1] = gather(i + 1)
        d_g[i].wait()
        d_o[i] = scatter(i)
        if i + 2 < nch:
            d_idx[i + 2] = idx_load(i + 2)
    if nch >= 2:
        d_o[nch - 2].wait()
    d_o[nch - 1].wait()


@functools.partial(jax.jit, static_argnames=("n", "d"))
def _gather(flat_idx, table, n, d):
    nch = n // (NW * CHUNK)
    mesh = plsc.VectorSubcoreMesh(core_axis_name="c", subcore_axis_name="s")
    return pl.kernel(
        functools.partial(_gather_body, nch),
        out_type=jax.ShapeDtypeStruct((n, d), jnp.float32),
        mesh=mesh,
        scratch_types=[
            pltpu.VMEM((CHUNK,), jnp.int32),
            pltpu.VMEM((CHUNK,), jnp.int32),
            pltpu.VMEM((CHUNK, d), jnp.float32),
            pltpu.VMEM((CHUNK, d), jnp.float32),
            pltpu.SemaphoreType.DMA,
            pltpu.SemaphoreType.DMA,
            pltpu.SemaphoreType.DMA,
            pltpu.SemaphoreType.DMA,
            pltpu.SemaphoreType.DMA,
            pltpu.SemaphoreType.DMA,
        ],
        compiler_params=pltpu.CompilerParams(use_tc_tiling_on_sc=False),
    )(flat_idx, table)


def kernel(action_idx, table):
    b, h = action_idx.shape
    n = b * h
    d = table.shape[1]
    flat_idx = action_idx.reshape(n).astype(jnp.int32)
    out = _gather(flat_idx, table, n, d)
    return out.reshape(b, h, d)


# speedup vs baseline: 1.5824x; 1.4239x over previous
import functools

import jax
import jax.numpy as jnp
from jax import lax
from jax.experimental import pallas as pl
from jax.experimental.pallas import tpu as pltpu
from jax.experimental.pallas import tpu_sc as plsc

NC = 2
NS = 16
NW = NC * NS

BLKB = 128
HP = 25
NBLK = 4
ROWS = 256
GBYTES = ROWS * 128 * 4
OBYTES = 2 * 32 * BLKB * 4


def _body(idx_hbm, table4_hbm, out3_hbm,
          idx_all, gidx, rbuf0, rbuf1, rows0, rows1, ov0, ov1, gsem, osem):
    wid = lax.axis_index("s") * NC + lax.axis_index("c")
    iota = lax.broadcasted_iota(jnp.int32, (16,), 0)

    def build(i, rbuf):
        h = 2 * i
        for half in range(2):
            for k in range(8):
                addr = (h + half) + 800 * k + 50 * iota
                v = plsc.load_gather(idx_all, [addr])
                gidx[pl.ds(128 * half + 16 * k, 16)] = v >> 2
                rbuf[pl.ds(128 * half + 16 * k, 16)] = (v & 3) * 32

    def transpose_half(rows_v, rbuf, out_v, half):
        def kstep(k, c):
            base = 128 * half + 16 * k
            rvec = base + iota
            rvals = rbuf[pl.ds(base, 16)]
            for f in range(32):
                vals = plsc.load_gather(rows_v, [rvec, rvals + f])
                out_v[f, pl.ds(16 * k, 16)] = vals
            return c
        lax.fori_loop(0, 8, kstep, 0, unroll=2)

    def process(rows_v, rbuf, h, b0):
        transpose_half(rows_v, rbuf, ov0, 0)
        pltpu.async_copy(ov0, out3_hbm.at[h, :, pl.ds(b0, BLKB)], osem)
        transpose_half(rows_v, rbuf, ov1, 1)
        pltpu.async_copy(ov1, out3_hbm.at[h + 1, :, pl.ds(b0, BLKB)], osem)

    def block(bi, carry):
        b0 = pl.multiple_of((4 * wid + bi) * BLKB, BLKB)
        pltpu.sync_copy(idx_hbm.at[pl.ds(b0 * 50, 50 * BLKB)], idx_all)

        build(0, rbuf0)
        pltpu.async_copy(table4_hbm.at[gidx], rows0, gsem)

        def wait_gather(i):
            pltpu.make_async_copy(
                table4_hbm.at[pl.ds(0, ROWS)], rows0, gsem).wait()

        def wait_out(i):
            pltpu.make_async_copy(
                ov0, out3_hbm.at[0, :, pl.ds(b0, BLKB)], osem).wait()
            pltpu.make_async_copy(
                ov1, out3_hbm.at[0, :, pl.ds(b0, BLKB)], osem).wait()

        def unit(i, carry2):
            p = lax.rem(i, 2)
            wait_gather(i)

            @pl.when(i < HP - 1)
            def _():
                @pl.when(p == 0)
                def _():
                    build(i + 1, rbuf1)
                    pltpu.async_copy(table4_hbm.at[gidx], rows1, gsem)

                @pl.when(p == 1)
                def _():
                    build(i + 1, rbuf0)
                    pltpu.async_copy(table4_hbm.at[gidx], rows0, gsem)

            @pl.when(i >= 1)
            def _():
                wait_out(i)

            h = 2 * i

            @pl.when(p == 0)
            def _():
                process(rows0, rbuf0, h, b0)

            @pl.when(p == 1)
            def _():
                process(rows1, rbuf1, h, b0)

            return carry2

        lax.fori_loop(0, HP, unit, 0)
        wait_out(HP)
        return carry

    lax.fori_loop(0, NBLK, block, 0)


@functools.partial(jax.jit, static_argnames=("n", "d"))
def _gather(flat_idx, table4, n, d):
    mesh = plsc.VectorSubcoreMesh(core_axis_name="c", subcore_axis_name="s")
    return pl.kernel(
        _body,
        out_type=jax.ShapeDtypeStruct((50, d, n // 50), jnp.float32),
        mesh=mesh,
        scratch_types=[
            pltpu.VMEM((50 * BLKB,), jnp.int32),
            pltpu.VMEM((ROWS,), jnp.int32),
            pltpu.VMEM((ROWS,), jnp.int32),
            pltpu.VMEM((ROWS,), jnp.int32),
            pltpu.VMEM((ROWS, 128), jnp.float32),
            pltpu.VMEM((ROWS, 128), jnp.float32),
            pltpu.VMEM((32, BLKB), jnp.float32),
            pltpu.VMEM((32, BLKB), jnp.float32),
            pltpu.SemaphoreType.DMA,
            pltpu.SemaphoreType.DMA,
        ],
        compiler_params=pltpu.CompilerParams(use_tc_tiling_on_sc=True, needs_layout_passes=False),
    )(flat_idx, table4)


def kernel(action_idx, table):
    b, h = action_idx.shape
    n = b * h
    d = table.shape[1]
    flat_idx = action_idx.reshape(n).astype(jnp.int32)
    table4 = table.reshape(table.shape[0] // 4, 128)
    out3 = _gather(flat_idx, table4, n, d)
    return jnp.transpose(out3, (2, 0, 1))
